# baseline (speedup 1.0000x reference)
import functools
import jax
import jax.numpy as jnp
from jax import lax
from jax.experimental import pallas as pl
from jax.experimental.pallas import tpu as pltpu
from jax.experimental.pallas import tpu_sc as plsc


def _vq_body(z_ref, e_ref, bi_ref, bd_ref, *, kb):
    k = pl.program_id(1)
    zb = z_ref[...]
    eb = e_ref[...]
    esq = jnp.sum(eb * eb, axis=1)
    s = esq[:, None] - 2.0 * lax.dot_general(
        eb, zb, (((1,), (1,)), ((), ())), preferred_element_type=jnp.float32)
    lmin = jnp.min(s, axis=0)
    iota = lax.broadcasted_iota(jnp.int32, s.shape, 0)
    larg = jnp.min(jnp.where(s == lmin[None, :], iota, jnp.int32(2 ** 30)),
                   axis=0) + k * kb

    @pl.when(k == 0)
    def _init():
        bd_ref[...] = jnp.full(bd_ref.shape, 3.4e38, jnp.float32)
        bi_ref[...] = jnp.zeros(bi_ref.shape, jnp.int32)

    prev_d = bd_ref[0, 0, :]
    prev_i = bi_ref[0, 0, :]
    upd = lmin < prev_d
    bd_ref[...] = jnp.where(upd, lmin, prev_d).reshape(bd_ref.shape)
    bi_ref[...] = jnp.where(upd, larg, prev_i).reshape(bi_ref.shape)


def _vq_argmin(zf, emb):
    n, c = zf.shape
    k = emb.shape[0]
    p, kb = 512, 512
    npb, nkb = n // p, k // kb
    bi, _ = pl.pallas_call(
        functools.partial(_vq_body, kb=kb),
        grid=(npb, nkb),
        in_specs=[pl.BlockSpec((p, c), lambda pi, ki: (pi, 0)),
                  pl.BlockSpec((kb, c), lambda pi, ki: (ki, 0))],
        out_specs=[pl.BlockSpec((1, 1, p), lambda pi, ki: (pi, 0, 0)),
                   pl.BlockSpec((1, 1, p), lambda pi, ki: (pi, 0, 0))],
        out_shape=[jax.ShapeDtypeStruct((npb, 1, p), jnp.int32),
                   jax.ShapeDtypeStruct((npb, 1, p), jnp.float32)],
    )(zf, emb)
    return bi.reshape(n)


def _sc_gather(table, idx):
    info = plsc.get_sparse_core_info()
    nc, ns = info.num_cores, info.num_subcores
    nw = nc * ns
    b = idx.shape[0]
    d = table.shape[1]
    bpw = b // nw
    mesh = plsc.VectorSubcoreMesh(core_axis_name="c", subcore_axis_name="s")

    @functools.partial(
        pl.kernel, mesh=mesh,
        out_type=jax.ShapeDtypeStruct((b, d), jnp.float32),
        scratch_types=[pltpu.VMEM((bpw,), jnp.int32),
                       pltpu.VMEM((bpw, d), jnp.float32),
                       pltpu.SemaphoreType.DMA],
    )
    def gk(table_hbm, idx_hbm, out_hbm, idx_v, rows_v, sem):
        wid = lax.axis_index("s") * nc + lax.axis_index("c")
        base = wid * bpw
        pltpu.sync_copy(idx_hbm.at[pl.ds(base, bpw)], idx_v)
        pltpu.async_copy(table_hbm.at[idx_v], rows_v, sem).wait()
        pltpu.sync_copy(rows_v, out_hbm.at[pl.ds(base, bpw)])

    return gk(table, idx)


def _mm_body(x_ref, w_ref, b_ref, o_ref):
    o_ref[...] = (jnp.dot(x_ref[...], w_ref[...],
                          preferred_element_type=jnp.float32) + b_ref[...])


def _mm_bias(x, w, bias):
    n, c = x.shape
    o = w.shape[1]
    p = 1024
    return pl.pallas_call(
        _mm_body,
        grid=(n // p,),
        in_specs=[pl.BlockSpec((p, c), lambda i: (i, 0)),
                  pl.BlockSpec((c, o), lambda i: (0, 0)),
                  pl.BlockSpec((1, o), lambda i: (0, 0))],
        out_specs=pl.BlockSpec((p, o), lambda i: (i, 0)),
        out_shape=jax.ShapeDtypeStruct((n, o), jnp.float32),
    )(x, w, bias.reshape(1, o))


def _conv_body(x0_ref, x1_ref, x2_ref, w_ref, b_ref, o_ref, *, relu):
    _, th, wd, c3 = x0_ref.shape
    o = o_ref.shape[3]
    m = th * wd
    acc = jnp.dot(x0_ref[0].reshape(m, c3), w_ref[0],
                  preferred_element_type=jnp.float32)
    acc = acc + jnp.dot(x1_ref[0].reshape(m, c3), w_ref[1],
                        preferred_element_type=jnp.float32)
    acc = acc + jnp.dot(x2_ref[0].reshape(m, c3), w_ref[2],
                        preferred_element_type=jnp.float32)
    acc = acc + b_ref[...]
    if relu:
        acc = jnp.maximum(acc, 0.0)
    o_ref[...] = acc.reshape(1, th, wd, o)


def _conv3x3(x, w_oihw, bias, relu, th):
    bsz, h, w, cin = x.shape
    o = w_oihw.shape[0]
    xp = jnp.pad(x, ((0, 0), (1, 1), (1, 1), (0, 0)))
    xc = jnp.concatenate(
        [xp[:, :, 0:w], xp[:, :, 1:w + 1], xp[:, :, 2:w + 2]], axis=3)
    x0, x1, x2 = xc[:, 0:h], xc[:, 1:h + 1], xc[:, 2:h + 2]
    wp = jnp.transpose(w_oihw, (2, 3, 1, 0)).reshape(3, 3 * cin, o)
    spec_x = pl.BlockSpec((1, th, w, 3 * cin), lambda bi, t: (bi, t, 0, 0))
    return pl.pallas_call(
        functools.partial(_conv_body, relu=relu),
        grid=(bsz, h // th),
        in_specs=[spec_x, spec_x, spec_x,
                  pl.BlockSpec((3, 3 * cin, o), lambda bi, t: (0, 0, 0)),
                  pl.BlockSpec((1, o), lambda bi, t: (0, 0))],
        out_specs=pl.BlockSpec((1, th, w, o), lambda bi, t: (bi, t, 0, 0)),
        out_shape=jax.ShapeDtypeStruct((bsz, h, w, o), jnp.float32),
    )(x0, x1, x2, wp, bias.reshape(1, o))


def _up2(x):
    return jnp.repeat(jnp.repeat(x, 2, axis=1), 2, axis=2)


def kernel(z, embedding_weight, pq_w, pq_b, w_in, b_in,
           w1, b1, w2, b2, w3, b3, w_out, b_out):
    bsz, c, h, w = z.shape
    zf = jnp.transpose(z, (0, 2, 3, 1)).reshape(-1, c)
    idx = _vq_argmin(zf, embedding_weight)
    zq = _sc_gather(embedding_weight, idx)
    quant = _mm_bias(zq, pq_w[:, :, 0, 0].T, pq_b)
    hh = quant.reshape(bsz, h, w, c)
    hh = _conv3x3(hh, w_in, b_in, relu=True, th=8)
    hh = _up2(hh)
    hh = _conv3x3(hh, w1, b1, relu=True, th=8)
    hh = _up2(hh)
    hh = _conv3x3(hh, w2, b2, relu=True, th=8)
    hh = _up2(hh)
    hh = _conv3x3(hh, w3, b3, relu=True, th=8)
    wo = jnp.pad(w_out, ((0, 125), (0, 0), (0, 0), (0, 0)))
    bo = jnp.pad(b_out, (0, 125))
    y = _conv3x3(hh, wo, bo, relu=False, th=8)[..., :3]
    return jnp.transpose(y, (0, 3, 1, 2))


# speedup vs baseline: 3.0801x; 3.0801x over previous
import functools
import jax
import jax.numpy as jnp
from jax import lax
from jax.experimental import pallas as pl
from jax.experimental.pallas import tpu as pltpu
from jax.experimental.pallas import tpu_sc as plsc


def _vq_body(z_ref, e_ref, bi_ref, bd_ref, *, kb):
    k = pl.program_id(1)
    zb = z_ref[...]
    eb = e_ref[...]
    esq = jnp.sum(eb * eb, axis=1)
    s = esq[:, None] - 2.0 * lax.dot_general(
        eb, zb, (((1,), (1,)), ((), ())), preferred_element_type=jnp.float32)
    lmin = jnp.min(s, axis=0)
    iota = lax.broadcasted_iota(jnp.int32, s.shape, 0)
    larg = jnp.min(jnp.where(s == lmin[None, :], iota, jnp.int32(2 ** 30)),
                   axis=0) + k * kb

    @pl.when(k == 0)
    def _init():
        bd_ref[...] = jnp.full(bd_ref.shape, 3.4e38, jnp.float32)
        bi_ref[...] = jnp.zeros(bi_ref.shape, jnp.int32)

    prev_d = bd_ref[0, 0, :]
    prev_i = bi_ref[0, 0, :]
    upd = lmin < prev_d
    bd_ref[...] = jnp.where(upd, lmin, prev_d).reshape(bd_ref.shape)
    bi_ref[...] = jnp.where(upd, larg, prev_i).reshape(bi_ref.shape)


def _vq_argmin(zf, emb):
    n, c = zf.shape
    k = emb.shape[0]
    p, kb = 512, 512
    npb, nkb = n // p, k // kb
    bi, _ = pl.pallas_call(
        functools.partial(_vq_body, kb=kb),
        grid=(npb, nkb),
        in_specs=[pl.BlockSpec((p, c), lambda pi, ki: (pi, 0)),
                  pl.BlockSpec((kb, c), lambda pi, ki: (ki, 0))],
        out_specs=[pl.BlockSpec((1, 1, p), lambda pi, ki: (pi, 0, 0)),
                   pl.BlockSpec((1, 1, p), lambda pi, ki: (pi, 0, 0))],
        out_shape=[jax.ShapeDtypeStruct((npb, 1, p), jnp.int32),
                   jax.ShapeDtypeStruct((npb, 1, p), jnp.float32)],
    )(zf, emb)
    return bi.reshape(n)


def _sc_gather(table, idx):
    info = plsc.get_sparse_core_info()
    nc, ns = info.num_cores, info.num_subcores
    nw = nc * ns
    b = idx.shape[0]
    d = table.shape[1]
    bpw = b // nw
    mesh = plsc.VectorSubcoreMesh(core_axis_name="c", subcore_axis_name="s")

    @functools.partial(
        pl.kernel, mesh=mesh,
        out_type=jax.ShapeDtypeStruct((b, d), jnp.float32),
        scratch_types=[pltpu.VMEM((bpw,), jnp.int32),
                       pltpu.VMEM((bpw, d), jnp.float32),
                       pltpu.SemaphoreType.DMA],
    )
    def gk(table_hbm, idx_hbm, out_hbm, idx_v, rows_v, sem):
        wid = lax.axis_index("s") * nc + lax.axis_index("c")
        base = wid * bpw
        pltpu.sync_copy(idx_hbm.at[pl.ds(base, bpw)], idx_v)
        pltpu.async_copy(table_hbm.at[idx_v], rows_v, sem).wait()
        pltpu.sync_copy(rows_v, out_hbm.at[pl.ds(base, bpw)])

    return gk(table, idx)


def _mm_body(x_ref, w_ref, b_ref, o_ref):
    o_ref[...] = (jnp.dot(x_ref[...], w_ref[...],
                          preferred_element_type=jnp.float32) + b_ref[...])


def _mm_bias(x, w, bias):
    n, c = x.shape
    o = w.shape[1]
    p = 1024
    return pl.pallas_call(
        _mm_body,
        grid=(n // p,),
        in_specs=[pl.BlockSpec((p, c), lambda i: (i, 0)),
                  pl.BlockSpec((c, o), lambda i: (0, 0)),
                  pl.BlockSpec((1, o), lambda i: (0, 0))],
        out_specs=pl.BlockSpec((p, o), lambda i: (i, 0)),
        out_shape=jax.ShapeDtypeStruct((n, o), jnp.float32),
    )(x, w, bias.reshape(1, o))


def _conv_body(x_ref, w_ref, b_ref, o_ref, *, relu):
    _, th, wd, o = o_ref.shape
    c3 = w_ref.shape[1]
    xs = x_ref[0]
    cat = jnp.concatenate(
        [xs[:, 0:wd], xs[:, 1:wd + 1], xs[:, 2:wd + 2]], axis=2)
    m = th * wd
    acc = jnp.dot(cat[0:th].reshape(m, c3), w_ref[0],
                  preferred_element_type=jnp.float32)
    acc = acc + jnp.dot(cat[1:th + 1].reshape(m, c3), w_ref[1],
                        preferred_element_type=jnp.float32)
    acc = acc + jnp.dot(cat[2:th + 2].reshape(m, c3), w_ref[2],
                        preferred_element_type=jnp.float32)
    acc = acc + b_ref[...]
    if relu:
        acc = jnp.maximum(acc, 0.0)
    o_ref[...] = acc.reshape(1, th, wd, o)


def _conv3x3(x, w_oihw, bias, relu, th):
    bsz, h, w, cin = x.shape
    o = w_oihw.shape[0]
    wp = w + 8
    xp = jnp.pad(x, ((0, 0), (1, 1), (1, wp - w - 1), (0, 0)))
    wk = jnp.transpose(w_oihw, (2, 3, 1, 0)).reshape(3, 3 * cin, o)
    return pl.pallas_call(
        functools.partial(_conv_body, relu=relu),
        grid=(bsz, h // th),
        in_specs=[pl.BlockSpec((pl.Element(1), pl.Element(th + 2),
                                pl.Element(wp), pl.Element(cin)),
                               lambda bi, t: (bi, t * th, 0, 0)),
                  pl.BlockSpec((3, 3 * cin, o), lambda bi, t: (0, 0, 0)),
                  pl.BlockSpec((1, o), lambda bi, t: (0, 0))],
        out_specs=pl.BlockSpec((1, th, w, o), lambda bi, t: (bi, t, 0, 0)),
        out_shape=jax.ShapeDtypeStruct((bsz, h, w, o), jnp.float32),
    )(xp, wk, bias.reshape(1, o))


def _up2(x):
    return jnp.repeat(jnp.repeat(x, 2, axis=1), 2, axis=2)


def kernel(z, embedding_weight, pq_w, pq_b, w_in, b_in,
           w1, b1, w2, b2, w3, b3, w_out, b_out):
    bsz, c, h, w = z.shape
    zf = jnp.transpose(z, (0, 2, 3, 1)).reshape(-1, c)
    idx = _vq_argmin(zf, embedding_weight)
    zq = _sc_gather(embedding_weight, idx)
    quant = _mm_bias(zq, pq_w[:, :, 0, 0].T, pq_b)
    hh = quant.reshape(bsz, h, w, c)
    hh = _conv3x3(hh, w_in, b_in, relu=True, th=8)
    hh = _up2(hh)
    hh = _conv3x3(hh, w1, b1, relu=True, th=8)
    hh = _up2(hh)
    hh = _conv3x3(hh, w2, b2, relu=True, th=8)
    hh = _up2(hh)
    hh = _conv3x3(hh, w3, b3, relu=True, th=8)
    wo = jnp.pad(w_out, ((0, 5), (0, 0), (0, 0), (0, 0)))
    bo = jnp.pad(b_out, (0, 5))
    y = _conv3x3(hh, wo, bo, relu=False, th=8)[..., :3]
    return jnp.transpose(y, (0, 3, 1, 2))


# speedup vs baseline: 5.0298x; 1.6330x over previous
import functools
import jax
import jax.numpy as jnp
from jax import lax
from jax.experimental import pallas as pl
from jax.experimental.pallas import tpu as pltpu
from jax.experimental.pallas import tpu_sc as plsc


def _vq_body(z_ref, e_ref, bi_ref, bd_ref, *, kb):
    k = pl.program_id(1)
    zb = z_ref[...]
    eb = e_ref[...]
    esq = jnp.sum(eb * eb, axis=1)
    s = esq[:, None] - 2.0 * lax.dot_general(
        eb.astype(jnp.bfloat16), zb.astype(jnp.bfloat16),
        (((1,), (1,)), ((), ())), preferred_element_type=jnp.float32)
    lmin = jnp.min(s, axis=0)
    iota = lax.broadcasted_iota(jnp.int32, s.shape, 0)
    larg = jnp.min(jnp.where(s == lmin[None, :], iota, jnp.int32(2 ** 30)),
                   axis=0) + k * kb

    @pl.when(k == 0)
    def _init():
        bd_ref[...] = jnp.full(bd_ref.shape, 3.4e38, jnp.float32)
        bi_ref[...] = jnp.zeros(bi_ref.shape, jnp.int32)

    prev_d = bd_ref[0, 0, :]
    prev_i = bi_ref[0, 0, :]
    upd = lmin < prev_d
    bd_ref[...] = jnp.where(upd, lmin, prev_d).reshape(bd_ref.shape)
    bi_ref[...] = jnp.where(upd, larg, prev_i).reshape(bi_ref.shape)


def _vq_argmin(zf, emb):
    n, c = zf.shape
    k = emb.shape[0]
    p, kb = 512, 512
    npb, nkb = n // p, k // kb
    bi, _ = pl.pallas_call(
        functools.partial(_vq_body, kb=kb),
        grid=(npb, nkb),
        in_specs=[pl.BlockSpec((p, c), lambda pi, ki: (pi, 0)),
                  pl.BlockSpec((kb, c), lambda pi, ki: (ki, 0))],
        out_specs=[pl.BlockSpec((1, 1, p), lambda pi, ki: (pi, 0, 0)),
                   pl.BlockSpec((1, 1, p), lambda pi, ki: (pi, 0, 0))],
        out_shape=[jax.ShapeDtypeStruct((npb, 1, p), jnp.int32),
                   jax.ShapeDtypeStruct((npb, 1, p), jnp.float32)],
    )(zf, emb)
    return bi.reshape(n)


def _sc_gather(table, idx):
    info = plsc.get_sparse_core_info()
    nc, ns = info.num_cores, info.num_subcores
    nw = nc * ns
    b = idx.shape[0]
    d = table.shape[1]
    bpw = b // nw
    mesh = plsc.VectorSubcoreMesh(core_axis_name="c", subcore_axis_name="s")

    @functools.partial(
        pl.kernel, mesh=mesh,
        out_type=jax.ShapeDtypeStruct((b, d), jnp.float32),
        scratch_types=[pltpu.VMEM((bpw,), jnp.int32),
                       pltpu.VMEM((bpw, d), jnp.float32),
                       pltpu.SemaphoreType.DMA],
    )
    def gk(table_hbm, idx_hbm, out_hbm, idx_v, rows_v, sem):
        wid = lax.axis_index("s") * nc + lax.axis_index("c")
        base = wid * bpw
        pltpu.sync_copy(idx_hbm.at[pl.ds(base, bpw)], idx_v)
        pltpu.async_copy(table_hbm.at[idx_v], rows_v, sem).wait()
        pltpu.sync_copy(rows_v, out_hbm.at[pl.ds(base, bpw)])

    return gk(table, idx)


def _mm_body(x_ref, w_ref, b_ref, o_ref):
    o_ref[...] = (jnp.dot(x_ref[...], w_ref[...],
                          preferred_element_type=jnp.float32) + b_ref[...])


def _mm_bias(x, w, bias):
    n, c = x.shape
    o = w.shape[1]
    p = 1024
    return pl.pallas_call(
        _mm_body,
        grid=(n // p,),
        in_specs=[pl.BlockSpec((p, c), lambda i: (i, 0)),
                  pl.BlockSpec((c, o), lambda i: (0, 0)),
                  pl.BlockSpec((1, o), lambda i: (0, 0))],
        out_specs=pl.BlockSpec((p, o), lambda i: (i, 0)),
        out_shape=jax.ShapeDtypeStruct((n, o), jnp.float32),
    )(x, w, bias.reshape(1, o))


def _conv_body(x_ref, w_ref, b_ref, o_ref, *, relu):
    _, th, wd, o = o_ref.shape
    c3 = w_ref.shape[1]
    xs = x_ref[0]
    cat = jnp.concatenate(
        [xs[:, 0:wd], xs[:, 1:wd + 1], xs[:, 2:wd + 2]], axis=2)
    m = th * wd
    acc = jnp.dot(cat[0:th].reshape(m, c3), w_ref[0],
                  preferred_element_type=jnp.float32)
    acc = acc + jnp.dot(cat[1:th + 1].reshape(m, c3), w_ref[1],
                        preferred_element_type=jnp.float32)
    acc = acc + jnp.dot(cat[2:th + 2].reshape(m, c3), w_ref[2],
                        preferred_element_type=jnp.float32)
    acc = acc + b_ref[...]
    if relu:
        acc = jnp.maximum(acc, 0.0)
    o_ref[...] = acc.reshape(1, th, wd, o)


def _conv3x3(x, w_oihw, bias, relu, th):
    bsz, h, w, cin = x.shape
    o = w_oihw.shape[0]
    wp = w + 8
    xp = jnp.pad(x, ((0, 0), (1, 1), (1, wp - w - 1), (0, 0)))
    wk = jnp.transpose(w_oihw, (2, 3, 1, 0)).reshape(3, 3 * cin, o)
    return pl.pallas_call(
        functools.partial(_conv_body, relu=relu),
        grid=(bsz, h // th),
        in_specs=[pl.BlockSpec((pl.Element(1), pl.Element(th + 2),
                                pl.Element(wp), pl.Element(cin)),
                               lambda bi, t: (bi, t * th, 0, 0)),
                  pl.BlockSpec((3, 3 * cin, o), lambda bi, t: (0, 0, 0)),
                  pl.BlockSpec((1, o), lambda bi, t: (0, 0))],
        out_specs=pl.BlockSpec((1, th, w, o), lambda bi, t: (bi, t, 0, 0)),
        out_shape=jax.ShapeDtypeStruct((bsz, h, w, o), jnp.float32),
    )(xp, wk, bias.reshape(1, o))


def _up2(x):
    return jnp.repeat(jnp.repeat(x, 2, axis=1), 2, axis=2)


def _upconv_body(x_ref, w_ref, b_ref, o_ref, *, relu):
    _, th2, wd2, o = o_ref.shape
    th, wd = th2 // 2, wd2 // 2
    c3 = w_ref.shape[1]
    c = c3 // 3
    xs = x_ref[0]
    mid = xs[1:th + 1]
    rep = jnp.concatenate(
        [xs[0:1],
         jnp.stack([mid, mid], axis=1).reshape(2 * th, xs.shape[1], c),
         xs[th + 1:th + 2]], axis=0)
    midc = rep[:, 1:wd + 1]
    repc = jnp.concatenate(
        [rep[:, 0:1],
         jnp.stack([midc, midc], axis=2).reshape(2 * th + 2, 2 * wd, c),
         rep[:, wd + 1:wd + 2]], axis=1)
    cat = jnp.concatenate(
        [repc[:, 0:wd2], repc[:, 1:wd2 + 1], repc[:, 2:wd2 + 2]], axis=2)
    m = th2 * wd2
    acc = jnp.dot(cat[0:th2].reshape(m, c3), w_ref[0],
                  preferred_element_type=jnp.float32)
    acc = acc + jnp.dot(cat[1:th2 + 1].reshape(m, c3), w_ref[1],
                        preferred_element_type=jnp.float32)
    acc = acc + jnp.dot(cat[2:th2 + 2].reshape(m, c3), w_ref[2],
                        preferred_element_type=jnp.float32)
    acc = acc + b_ref[...]
    if relu:
        acc = jnp.maximum(acc, 0.0)
    o_ref[...] = acc.reshape(1, th2, wd2, o)


def _upconv3x3(x, w_oihw, bias, relu, th):
    bsz, h, w, cin = x.shape
    o = w_oihw.shape[0]
    wp = w + 8
    xp = jnp.pad(x, ((0, 0), (1, 1), (1, wp - w - 1), (0, 0)))
    wk = jnp.transpose(w_oihw, (2, 3, 1, 0)).reshape(3, 3 * cin, o)
    return pl.pallas_call(
        functools.partial(_upconv_body, relu=relu),
        grid=(bsz, h // th),
        in_specs=[pl.BlockSpec((pl.Element(1), pl.Element(th + 2),
                                pl.Element(wp), pl.Element(cin)),
                               lambda bi, t: (bi, t * th, 0, 0)),
                  pl.BlockSpec((3, 3 * cin, o), lambda bi, t: (0, 0, 0)),
                  pl.BlockSpec((1, o), lambda bi, t: (0, 0))],
        out_specs=pl.BlockSpec((1, 2 * th, 2 * w, o),
                               lambda bi, t: (bi, t, 0, 0)),
        out_shape=jax.ShapeDtypeStruct((bsz, 2 * h, 2 * w, o), jnp.float32),
    )(xp, wk, bias.reshape(1, o))


def _upconv_out_body(x_ref, w3_ref, b3_ref, wo_ref, bo_ref, o_ref, *, nt):
    t = pl.program_id(1)
    _, th2, wd2, oo = o_ref.shape
    th, wd = th2 // 2, wd2 // 2
    c3 = w3_ref.shape[1]
    c = c3 // 3
    o3 = w3_ref.shape[2]
    xs = x_ref[0]
    rep = jnp.stack([xs, xs], axis=1).reshape(2 * th + 4, xs.shape[1], c)
    midc = rep[:, 1:wd + 1]
    repc = jnp.concatenate(
        [rep[:, 0:1],
         jnp.stack([midc, midc], axis=2).reshape(2 * th + 4, 2 * wd, c),
         rep[:, wd + 1:wd + 2]], axis=1)
    cat = jnp.concatenate(
        [repc[:, 0:wd2], repc[:, 1:wd2 + 1], repc[:, 2:wd2 + 2]], axis=2)
    m3 = (th2 + 2) * wd2
    h3 = jnp.dot(cat[0:th2 + 2].reshape(m3, c3), w3_ref[0],
                 preferred_element_type=jnp.float32)
    h3 = h3 + jnp.dot(cat[1:th2 + 3].reshape(m3, c3), w3_ref[1],
                      preferred_element_type=jnp.float32)
    h3 = h3 + jnp.dot(cat[2:th2 + 4].reshape(m3, c3), w3_ref[2],
                      preferred_element_type=jnp.float32)
    h3 = jnp.maximum(h3 + b3_ref[...], 0.0).reshape(th2 + 2, wd2, o3)
    ridx = lax.broadcasted_iota(jnp.int32, (th2 + 2, 1, 1), 0)
    edge = ((ridx == 0) & (t == 0)) | ((ridx == th2 + 1) & (t == nt - 1))
    h3 = jnp.where(edge, 0.0, h3)
    zc = jnp.zeros((th2 + 2, 1, o3), jnp.float32)
    h3p = jnp.concatenate([zc, h3, zc], axis=1)
    cato = jnp.concatenate(
        [h3p[:, 0:wd2], h3p[:, 1:wd2 + 1], h3p[:, 2:wd2 + 2]], axis=2)
    mo = th2 * wd2
    acc = jnp.dot(cato[0:th2].reshape(mo, 3 * o3), wo_ref[0],
                  preferred_element_type=jnp.float32)
    acc = acc + jnp.dot(cato[1:th2 + 1].reshape(mo, 3 * o3), wo_ref[1],
                        preferred_element_type=jnp.float32)
    acc = acc + jnp.dot(cato[2:th2 + 2].reshape(mo, 3 * o3), wo_ref[2],
                        preferred_element_type=jnp.float32)
    acc = acc + bo_ref[...]
    o_ref[...] = acc.reshape(1, th2, wd2, oo)


def _upconv_out(x, w3_oihw, b3_, wo_oihw, bo_, th):
    bsz, h, w, cin = x.shape
    o3 = w3_oihw.shape[0]
    oo = wo_oihw.shape[0]
    wp = w + 8
    xp = jnp.pad(x, ((0, 0), (1, 1), (1, wp - w - 1), (0, 0)))
    w3k = jnp.transpose(w3_oihw, (2, 3, 1, 0)).reshape(3, 3 * cin, o3)
    wok = jnp.transpose(wo_oihw, (2, 3, 1, 0)).reshape(3, 3 * o3, oo)
    nt = h // th
    return pl.pallas_call(
        functools.partial(_upconv_out_body, nt=nt),
        grid=(bsz, nt),
        in_specs=[pl.BlockSpec((pl.Element(1), pl.Element(th + 2),
                                pl.Element(wp), pl.Element(cin)),
                               lambda bi, t: (bi, t * th, 0, 0)),
                  pl.BlockSpec((3, 3 * cin, o3), lambda bi, t: (0, 0, 0)),
                  pl.BlockSpec((1, o3), lambda bi, t: (0, 0)),
                  pl.BlockSpec((3, 3 * o3, oo), lambda bi, t: (0, 0, 0)),
                  pl.BlockSpec((1, oo), lambda bi, t: (0, 0))],
        out_specs=pl.BlockSpec((1, 2 * th, 2 * w, oo),
                               lambda bi, t: (bi, t, 0, 0)),
        out_shape=jax.ShapeDtypeStruct((bsz, 2 * h, 2 * w, oo), jnp.float32),
    )(xp, w3k, b3_.reshape(1, o3), wok, bo_.reshape(1, oo))


def kernel(z, embedding_weight, pq_w, pq_b, w_in, b_in,
           w1, b1, w2, b2, w3, b3, w_out, b_out):
    bsz, c, h, w = z.shape
    zf = jnp.transpose(z, (0, 2, 3, 1)).reshape(-1, c)
    idx = _vq_argmin(zf, embedding_weight)
    zq = _sc_gather(embedding_weight, idx)
    quant = _mm_bias(zq, pq_w[:, :, 0, 0].T, pq_b)
    hh = quant.reshape(bsz, h, w, c)
    hh = _conv3x3(hh, w_in, b_in, relu=True, th=8)
    hh = _upconv3x3(hh, w1, b1, relu=True, th=8)
    hh = _upconv3x3(hh, w2, b2, relu=True, th=8)
    wo = jnp.pad(w_out, ((0, 5), (0, 0), (0, 0), (0, 0)))
    bo = jnp.pad(b_out, (0, 5))
    y = _upconv_out(hh, w3, b3, wo, bo, th=8)[..., :3]
    return jnp.transpose(y, (0, 3, 1, 2))


# speedup vs baseline: 5.3016x; 1.0540x over previous
import functools
import jax
import jax.numpy as jnp
from jax import lax
from jax.experimental import pallas as pl
from jax.experimental.pallas import tpu as pltpu
from jax.experimental.pallas import tpu_sc as plsc


def _vq_body(z_ref, e_ref, bi_ref, bd_ref, *, kb):
    k = pl.program_id(1)
    zb = z_ref[0]
    eb = e_ref[...]
    esq = jnp.sum(eb * eb, axis=1)
    s = esq[:, None] - 2.0 * lax.dot_general(
        eb.astype(jnp.bfloat16), zb.astype(jnp.bfloat16),
        (((1,), (0,)), ((), ())), preferred_element_type=jnp.float32)
    lmin = jnp.min(s, axis=0)
    iota = lax.broadcasted_iota(jnp.int32, s.shape, 0)
    larg = jnp.min(jnp.where(s == lmin[None, :], iota, jnp.int32(2 ** 30)),
                   axis=0) + k * kb

    @pl.when(k == 0)
    def _init():
        bd_ref[...] = jnp.full(bd_ref.shape, 3.4e38, jnp.float32)
        bi_ref[...] = jnp.zeros(bi_ref.shape, jnp.int32)

    prev_d = bd_ref[0, 0, :]
    prev_i = bi_ref[0, 0, :]
    upd = lmin < prev_d
    bd_ref[...] = jnp.where(upd, lmin, prev_d).reshape(bd_ref.shape)
    bi_ref[...] = jnp.where(upd, larg, prev_i).reshape(bi_ref.shape)


def _vq_argmin(zr, emb):
    bsz, c, hw = zr.shape
    n = bsz * hw
    k = emb.shape[0]
    p, kb = hw, 512
    npb, nkb = bsz, k // kb
    bi, _ = pl.pallas_call(
        functools.partial(_vq_body, kb=kb),
        grid=(npb, nkb),
        in_specs=[pl.BlockSpec((1, c, p), lambda pi, ki: (pi, 0, 0)),
                  pl.BlockSpec((kb, c), lambda pi, ki: (ki, 0))],
        out_specs=[pl.BlockSpec((1, 1, p), lambda pi, ki: (pi, 0, 0)),
                   pl.BlockSpec((1, 1, p), lambda pi, ki: (pi, 0, 0))],
        out_shape=[jax.ShapeDtypeStruct((npb, 1, p), jnp.int32),
                   jax.ShapeDtypeStruct((npb, 1, p), jnp.float32)],
    )(zr, emb)
    return bi.reshape(n)


def _sc_gather(table, idx):
    info = plsc.get_sparse_core_info()
    nc, ns = info.num_cores, info.num_subcores
    nw = nc * ns
    b = idx.shape[0]
    d = table.shape[1]
    bpw = b // nw
    mesh = plsc.VectorSubcoreMesh(core_axis_name="c", subcore_axis_name="s")

    @functools.partial(
        pl.kernel, mesh=mesh,
        out_type=jax.ShapeDtypeStruct((b, d), jnp.float32),
        scratch_types=[pltpu.VMEM((bpw,), jnp.int32),
                       pltpu.VMEM((bpw, d), jnp.float32),
                       pltpu.SemaphoreType.DMA],
    )
    def gk(table_hbm, idx_hbm, out_hbm, idx_v, rows_v, sem):
        wid = lax.axis_index("s") * nc + lax.axis_index("c")
        base = wid * bpw
        pltpu.sync_copy(idx_hbm.at[pl.ds(base, bpw)], idx_v)
        pltpu.async_copy(table_hbm.at[idx_v], rows_v, sem).wait()
        pltpu.sync_copy(rows_v, out_hbm.at[pl.ds(base, bpw)])

    return gk(table, idx)


def _mm_body(x_ref, w_ref, b_ref, o_ref):
    o_ref[...] = (jnp.dot(x_ref[...], w_ref[...],
                          preferred_element_type=jnp.float32) + b_ref[...])


def _mm_bias(x, w, bias):
    n, c = x.shape
    o = w.shape[1]
    p = 1024
    return pl.pallas_call(
        _mm_body,
        grid=(n // p,),
        in_specs=[pl.BlockSpec((p, c), lambda i: (i, 0)),
                  pl.BlockSpec((c, o), lambda i: (0, 0)),
                  pl.BlockSpec((1, o), lambda i: (0, 0))],
        out_specs=pl.BlockSpec((p, o), lambda i: (i, 0)),
        out_shape=jax.ShapeDtypeStruct((n, o), jnp.float32),
    )(x, w, bias.reshape(1, o))


def _conv_body(x_ref, w_ref, b_ref, o_ref, *, relu):
    _, th, wd, o = o_ref.shape
    c3 = w_ref.shape[1]
    xs = x_ref[0]
    cat = jnp.concatenate(
        [xs[:, 0:wd], xs[:, 1:wd + 1], xs[:, 2:wd + 2]], axis=2)
    m = th * wd
    acc = jnp.dot(cat[0:th].reshape(m, c3), w_ref[0],
                  preferred_element_type=jnp.float32)
    acc = acc + jnp.dot(cat[1:th + 1].reshape(m, c3), w_ref[1],
                        preferred_element_type=jnp.float32)
    acc = acc + jnp.dot(cat[2:th + 2].reshape(m, c3), w_ref[2],
                        preferred_element_type=jnp.float32)
    acc = acc + b_ref[...]
    if relu:
        acc = jnp.maximum(acc, 0.0)
    o_ref[...] = acc.reshape(1, th, wd, o)


def _conv3x3(x, w_oihw, bias, relu, th):
    bsz, h, w, cin = x.shape
    o = w_oihw.shape[0]
    wp = w + 8
    xp = jnp.pad(x, ((0, 0), (1, 1), (1, wp - w - 1), (0, 0)))
    wk = jnp.transpose(w_oihw, (2, 3, 1, 0)).reshape(3, 3 * cin, o)
    return pl.pallas_call(
        functools.partial(_conv_body, relu=relu),
        grid=(bsz, h // th),
        in_specs=[pl.BlockSpec((pl.Element(1), pl.Element(th + 2),
                                pl.Element(wp), pl.Element(cin)),
                               lambda bi, t: (bi, t * th, 0, 0)),
                  pl.BlockSpec((3, 3 * cin, o), lambda bi, t: (0, 0, 0)),
                  pl.BlockSpec((1, o), lambda bi, t: (0, 0))],
        out_specs=pl.BlockSpec((1, th, w, o), lambda bi, t: (bi, t, 0, 0)),
        out_shape=jax.ShapeDtypeStruct((bsz, h, w, o), jnp.float32),
    )(xp, wk, bias.reshape(1, o))


def _up2(x):
    return jnp.repeat(jnp.repeat(x, 2, axis=1), 2, axis=2)


def _upconv_body(x_ref, w_ref, b_ref, o_ref, *, relu):
    _, th2, wd2, o = o_ref.shape
    th, wd = th2 // 2, wd2 // 2
    c3 = w_ref.shape[1]
    c = c3 // 3
    xs = x_ref[0]
    mid = xs[1:th + 1]
    rep = jnp.concatenate(
        [xs[0:1],
         jnp.stack([mid, mid], axis=1).reshape(2 * th, xs.shape[1], c),
         xs[th + 1:th + 2]], axis=0)
    midc = rep[:, 1:wd + 1]
    repc = jnp.concatenate(
        [rep[:, 0:1],
         jnp.stack([midc, midc], axis=2).reshape(2 * th + 2, 2 * wd, c),
         rep[:, wd + 1:wd + 2]], axis=1)
    cat = jnp.concatenate(
        [repc[:, 0:wd2], repc[:, 1:wd2 + 1], repc[:, 2:wd2 + 2]], axis=2)
    m = th2 * wd2
    acc = jnp.dot(cat[0:th2].reshape(m, c3), w_ref[0],
                  preferred_element_type=jnp.float32)
    acc = acc + jnp.dot(cat[1:th2 + 1].reshape(m, c3), w_ref[1],
                        preferred_element_type=jnp.float32)
    acc = acc + jnp.dot(cat[2:th2 + 2].reshape(m, c3), w_ref[2],
                        preferred_element_type=jnp.float32)
    acc = acc + b_ref[...]
    if relu:
        acc = jnp.maximum(acc, 0.0)
    o_ref[...] = acc.reshape(1, th2, wd2, o)


def _upconv3x3(x, w_oihw, bias, relu, th):
    bsz, h, w, cin = x.shape
    o = w_oihw.shape[0]
    wp = w + 8
    xp = jnp.pad(x, ((0, 0), (1, 1), (1, wp - w - 1), (0, 0)))
    wk = jnp.transpose(w_oihw, (2, 3, 1, 0)).reshape(3, 3 * cin, o)
    return pl.pallas_call(
        functools.partial(_upconv_body, relu=relu),
        grid=(bsz, h // th),
        in_specs=[pl.BlockSpec((pl.Element(1), pl.Element(th + 2),
                                pl.Element(wp), pl.Element(cin)),
                               lambda bi, t: (bi, t * th, 0, 0)),
                  pl.BlockSpec((3, 3 * cin, o), lambda bi, t: (0, 0, 0)),
                  pl.BlockSpec((1, o), lambda bi, t: (0, 0))],
        out_specs=pl.BlockSpec((1, 2 * th, 2 * w, o),
                               lambda bi, t: (bi, t, 0, 0)),
        out_shape=jax.ShapeDtypeStruct((bsz, 2 * h, 2 * w, o), jnp.float32),
    )(xp, wk, bias.reshape(1, o))


def _upconv_out_body(x_ref, w3_ref, b3_ref, wo_ref, bo_ref, o_ref, *, nt):
    t = pl.program_id(1)
    _, th2, wd2, oo = o_ref.shape
    th, wd = th2 // 2, wd2 // 2
    c3 = w3_ref.shape[1]
    c = c3 // 3
    o3 = w3_ref.shape[2]
    xs = x_ref[0]
    rep = jnp.stack([xs, xs], axis=1).reshape(2 * th + 4, xs.shape[1], c)
    midc = rep[:, 1:wd + 1]
    repc = jnp.concatenate(
        [rep[:, 0:1],
         jnp.stack([midc, midc], axis=2).reshape(2 * th + 4, 2 * wd, c),
         rep[:, wd + 1:wd + 2]], axis=1)
    cat = jnp.concatenate(
        [repc[:, 0:wd2], repc[:, 1:wd2 + 1], repc[:, 2:wd2 + 2]], axis=2)
    m3 = (th2 + 2) * wd2
    h3 = jnp.dot(cat[0:th2 + 2].reshape(m3, c3), w3_ref[0],
                 preferred_element_type=jnp.float32)
    h3 = h3 + jnp.dot(cat[1:th2 + 3].reshape(m3, c3), w3_ref[1],
                      preferred_element_type=jnp.float32)
    h3 = h3 + jnp.dot(cat[2:th2 + 4].reshape(m3, c3), w3_ref[2],
                      preferred_element_type=jnp.float32)
    h3 = jnp.maximum(h3 + b3_ref[...], 0.0).reshape(th2 + 2, wd2, o3)
    ridx = lax.broadcasted_iota(jnp.int32, (th2 + 2, 1, 1), 0)
    edge = ((ridx == 0) & (t == 0)) | ((ridx == th2 + 1) & (t == nt - 1))
    h3 = jnp.where(edge, 0.0, h3)
    zc = jnp.zeros((th2 + 2, 1, o3), jnp.float32)
    h3p = jnp.concatenate([zc, h3, zc], axis=1)
    cato = jnp.concatenate(
        [h3p[:, 0:wd2], h3p[:, 1:wd2 + 1], h3p[:, 2:wd2 + 2]], axis=2)
    mo = th2 * wd2
    acc = jnp.dot(cato[0:th2].reshape(mo, 3 * o3), wo_ref[0],
                  preferred_element_type=jnp.float32)
    acc = acc + jnp.dot(cato[1:th2 + 1].reshape(mo, 3 * o3), wo_ref[1],
                        preferred_element_type=jnp.float32)
    acc = acc + jnp.dot(cato[2:th2 + 2].reshape(mo, 3 * o3), wo_ref[2],
                        preferred_element_type=jnp.float32)
    acc = acc + bo_ref[...]
    o_ref[...] = acc.reshape(1, th2, wd2, oo)


def _upconv_out(x, w3_oihw, b3_, wo_oihw, bo_, th):
    bsz, h, w, cin = x.shape
    o3 = w3_oihw.shape[0]
    oo = wo_oihw.shape[0]
    wp = w + 8
    xp = jnp.pad(x, ((0, 0), (1, 1), (1, wp - w - 1), (0, 0)))
    w3k = jnp.transpose(w3_oihw, (2, 3, 1, 0)).reshape(3, 3 * cin, o3)
    wok = jnp.transpose(wo_oihw, (2, 3, 1, 0)).reshape(3, 3 * o3, oo)
    nt = h // th
    return pl.pallas_call(
        functools.partial(_upconv_out_body, nt=nt),
        grid=(bsz, nt),
        in_specs=[pl.BlockSpec((pl.Element(1), pl.Element(th + 2),
                                pl.Element(wp), pl.Element(cin)),
                               lambda bi, t: (bi, t * th, 0, 0)),
                  pl.BlockSpec((3, 3 * cin, o3), lambda bi, t: (0, 0, 0)),
                  pl.BlockSpec((1, o3), lambda bi, t: (0, 0)),
                  pl.BlockSpec((3, 3 * o3, oo), lambda bi, t: (0, 0, 0)),
                  pl.BlockSpec((1, oo), lambda bi, t: (0, 0))],
        out_specs=pl.BlockSpec((1, 2 * th, 2 * w, oo),
                               lambda bi, t: (bi, t, 0, 0)),
        out_shape=jax.ShapeDtypeStruct((bsz, 2 * h, 2 * w, oo), jnp.float32),
    )(xp, w3k, b3_.reshape(1, o3), wok, bo_.reshape(1, oo))


def kernel(z, embedding_weight, pq_w, pq_b, w_in, b_in,
           w1, b1, w2, b2, w3, b3, w_out, b_out):
    bsz, c, h, w = z.shape
    idx = _vq_argmin(z.reshape(bsz, c, h * w), embedding_weight)
    zq = _sc_gather(embedding_weight, idx)
    quant = _mm_bias(zq, pq_w[:, :, 0, 0].T, pq_b)
    hh = quant.reshape(bsz, h, w, c)
    hh = _conv3x3(hh, w_in, b_in, relu=True, th=8)
    hh = _upconv3x3(hh, w1, b1, relu=True, th=8)
    hh = _upconv3x3(hh, w2, b2, relu=True, th=8)
    wo = jnp.pad(w_out, ((0, 5), (0, 0), (0, 0), (0, 0)))
    bo = jnp.pad(b_out, (0, 5))
    y = _upconv_out(hh, w3, b3, wo, bo, th=8)[..., :3]
    return jnp.transpose(y, (0, 3, 1, 2))
